# baseline (speedup 1.0000x reference)
import functools

import jax
import jax.numpy as jnp
from jax import lax
from jax.experimental import pallas as pl
from jax.experimental.pallas import tpu as pltpu
from jax.experimental.pallas import tpu_sc as plsc

N = 10000
NP = 10240
D = 128
E = 320000
NW = 32
EPW = E // NW
C = 80
NCHUNK = EPW // C
ROWS_PT = NP // 16


def _prologue_body(x_ref, wt_ref, al_ref, ar_ref, z_ref, s_ref, d_ref):
    z = jnp.dot(x_ref[...], wt_ref[...], preferred_element_type=jnp.float32)
    z_ref[...] = z
    s_ref[...] = jnp.sum(z * al_ref[...][None, :], axis=1)
    d_ref[...] = jnp.sum(z * ar_ref[...][None, :], axis=1)


NB = 2


def _sc_body(z_hbm, s_hbm, d_hbm, src_hbm, dst_hbm, znd_hbm, zn_hbm,
             ph_hbm, pd_hbm,
             s_v, d_v, src_r, dst_r, w_c, rows, h_sh, den_sh,
             gsem, hsem, wsem, isem, jsem):
    cid = lax.axis_index("c")
    sid = lax.axis_index("s")
    wid = cid * 16 + sid
    pltpu.sync_copy(s_hbm, s_v)
    pltpu.sync_copy(d_hbm, d_v)
    pltpu.sync_copy(znd_hbm, h_sh.at[pl.ds(sid * ROWS_PT, ROWS_PT)])
    pltpu.sync_copy(zn_hbm, den_sh.at[pl.ds(sid * ROWS_PT, ROWS_PT)])
    plsc.subcore_barrier()

    ebase = wid * EPW
    pltpu.sync_copy(src_hbm.at[pl.ds(ebase, C)], src_r.at[0])
    pltpu.sync_copy(dst_hbm.at[pl.ds(ebase, C)], dst_r.at[0])
    pltpu.async_copy(src_hbm.at[pl.ds(ebase + C, C)], src_r.at[1], isem.at[1])
    pltpu.async_copy(dst_hbm.at[pl.ds(ebase + C, C)], dst_r.at[1], jsem.at[1])
    pltpu.async_copy(z_hbm.at[src_r.at[0]], rows.at[0], gsem.at[0])

    def slot(t, b, ib, has_next, has_next2):
        bn = 1 - b
        ibn = (ib + 1) % 3
        ibp = (ib + 2) % 3
        pltpu.make_async_copy(z_hbm.at[src_r.at[ib]], rows.at[b],
                              gsem.at[b]).wait()
        if has_next:
            pltpu.make_async_copy(src_hbm.at[pl.ds(ebase + (t + 1) * C, C)], src_r.at[ibn],
                                  isem.at[ibn]).wait()
            pltpu.make_async_copy(dst_hbm.at[pl.ds(ebase + (t + 1) * C, C)], dst_r.at[ibn],
                                  jsem.at[ibn]).wait()

            @pl.when(t >= 1)
            def _():
                pltpu.make_async_copy(rows.at[bn], h_sh.at[dst_r.at[ibp]],
                                      hsem.at[bn]).wait()
                pltpu.make_async_copy(w_c.at[bn], den_sh.at[dst_r.at[ibp]],
                                      wsem.at[bn]).wait()

            pltpu.async_copy(z_hbm.at[src_r.at[ibn]], rows.at[bn],
                             gsem.at[bn])
        for k in range(C // 16):
            si = src_r[ib, pl.ds(k * 16, 16)]
            di = dst_r[ib, pl.ds(k * 16, 16)]
            e = plsc.load_gather(s_v, [si]) + plsc.load_gather(d_v, [di])
            e = jnp.where(e >= 0.0, e, e * 0.01)
            w_c[b, pl.ds(k * 16, 16)] = jnp.exp(e)

        @plsc.parallel_loop(0, C // 16)
        def _(k):
            w16 = w_c[b, pl.ds(k * 16, 16)]
            base16 = k * 16
            for lane in range(16):
                ws = w16[lane]
                i = base16 + lane
                for j in range(D // 16):
                    rows[b, i, pl.ds(j * 16, 16)] = (
                        rows[b, i, pl.ds(j * 16, 16)] * ws)
        pltpu.async_copy(rows.at[b], h_sh.at[dst_r.at[ib]], hsem.at[b],
                         add=True)
        pltpu.async_copy(w_c.at[b], den_sh.at[dst_r.at[ib]], wsem.at[b],
                         add=True)
        if has_next2:
            pltpu.async_copy(src_hbm.at[pl.ds(ebase + (t + 2) * C, C)], src_r.at[ibp],
                             isem.at[ibp])
            pltpu.async_copy(dst_hbm.at[pl.ds(ebase + (t + 2) * C, C)], dst_r.at[ibp],
                             jsem.at[ibp])

    def group_body(g, carry):
        t0 = g * 6
        for u in range(6):
            slot(t0 + u, u % 2, u % 3, True, True)
        return carry

    lax.fori_loop(0, NCHUNK // 6, group_body, 0)
    for t in range(NCHUNK // 6 * 6, NCHUNK):
        slot(t, t % 2, t % 3, t + 1 <= NCHUNK - 1, t + 2 <= NCHUNK - 1)
    for t in (NCHUNK - 2, NCHUNK - 1):
        b, ib = t % 2, t % 3
        pltpu.make_async_copy(rows.at[b], h_sh.at[dst_r.at[ib]],
                              hsem.at[b]).wait()
        pltpu.make_async_copy(w_c.at[b], den_sh.at[dst_r.at[ib]],
                              wsem.at[b]).wait()
    plsc.subcore_barrier()

    pltpu.sync_copy(h_sh.at[pl.ds(sid * ROWS_PT, ROWS_PT)],
                    ph_hbm.at[cid, pl.ds(sid * ROWS_PT, ROWS_PT)])

    @pl.when(sid == 0)
    def _():
        pltpu.sync_copy(den_sh, pd_hbm.at[cid])


def _epilogue_body(ph0_ref, ph1_ref, den_ref, out_ref):
    den = den_ref[0, :] + den_ref[1, :]
    rec = (1.0 / jnp.where(den > 0.0, den, 1.0)).reshape(-1, 1)
    out_ref[...] = (ph0_ref[0] + ph1_ref[0]) * rec


def kernel(x, edge_index, W_fc, W_attn):
    x_pad = jnp.pad(x, ((0, NP - N), (0, 0)))
    wt = W_fc.T
    al = W_attn[0, :D]
    ar = W_attn[0, D:]

    blk = 1024
    grid = NP // blk
    z, s, d = pl.pallas_call(
        _prologue_body,
        grid=(grid,),
        in_specs=[
            pl.BlockSpec((blk, D), lambda i: (i, 0)),
            pl.BlockSpec((D, D), lambda i: (0, 0)),
            pl.BlockSpec((D,), lambda i: (0,)),
            pl.BlockSpec((D,), lambda i: (0,)),
        ],
        out_specs=[
            pl.BlockSpec((blk, D), lambda i: (i, 0)),
            pl.BlockSpec((blk,), lambda i: (i,)),
            pl.BlockSpec((blk,), lambda i: (i,)),
        ],
        out_shape=[
            jax.ShapeDtypeStruct((NP, D), jnp.float32),
            jax.ShapeDtypeStruct((NP,), jnp.float32),
            jax.ShapeDtypeStruct((NP,), jnp.float32),
        ],
    )(x_pad, wt, al, ar)

    src = edge_index[0]
    dst = edge_index[1]
    zeros_nd = jnp.zeros((ROWS_PT, D), jnp.float32)
    zeros_n = jnp.zeros((ROWS_PT,), jnp.float32)

    mesh = plsc.VectorSubcoreMesh(core_axis_name="c", subcore_axis_name="s")
    sc = pl.kernel(
        _sc_body,
        out_type=[
            jax.ShapeDtypeStruct((2, NP, D), jnp.float32),
            jax.ShapeDtypeStruct((2, NP), jnp.float32),
        ],
        mesh=mesh,
        scratch_types=[
            pltpu.VMEM((NP,), jnp.float32),
            pltpu.VMEM((NP,), jnp.float32),
            pltpu.VMEM((3, C), jnp.int32),
            pltpu.VMEM((3, C), jnp.int32),
            pltpu.VMEM((NB, C), jnp.float32),
            pltpu.VMEM((NB, C, D), jnp.float32),
            pltpu.VMEM_SHARED((NP, D), jnp.float32),
            pltpu.VMEM_SHARED((NP,), jnp.float32),
            pltpu.SemaphoreType.DMA((NB,)),
            pltpu.SemaphoreType.DMA((NB,)),
            pltpu.SemaphoreType.DMA((NB,)),
            pltpu.SemaphoreType.DMA((3,)),
            pltpu.SemaphoreType.DMA((3,)),
        ],
        compiler_params=pltpu.CompilerParams(needs_layout_passes=False),
    )
    ph, pd = sc(z, s, d, src, dst, zeros_nd, zeros_n)

    h_pad = pl.pallas_call(
        _epilogue_body,
        grid=(grid,),
        in_specs=[
            pl.BlockSpec((1, blk, D), lambda i: (0, i, 0)),
            pl.BlockSpec((1, blk, D), lambda i: (1, i, 0)),
            pl.BlockSpec((2, blk), lambda i: (0, i)),
        ],
        out_specs=pl.BlockSpec((blk, D), lambda i: (i, 0)),
        out_shape=jax.ShapeDtypeStruct((NP, D), jnp.float32),
    )(ph, ph, pd)

    return h_pad[:N]


# speedup vs baseline: 1.2511x; 1.2511x over previous
import functools

import jax
import jax.numpy as jnp
from jax import lax
from jax.experimental import pallas as pl
from jax.experimental.pallas import tpu as pltpu
from jax.experimental.pallas import tpu_sc as plsc

N = 10000
NP = 10240
D = 128
E = 320000
NW = 32
EPW = E // NW
C = 80
NCHUNK = EPW // C
ROWS_PT = NP // 16


def _prologue_body(x_ref, w_ref, wa_ref, z_ref, s_ref, d_ref):
    z = lax.dot_general(x_ref[...], w_ref[...], (((1,), (1,)), ((), ())),
                        preferred_element_type=jnp.float32)
    z_ref[...] = z
    s_ref[...] = jnp.sum(z * wa_ref[0, :D][None, :], axis=1)
    d_ref[...] = jnp.sum(z * wa_ref[0, D:][None, :], axis=1)


NB = 2


def _sc_body(z_hbm, s_hbm, d_hbm, src_hbm, dst_hbm, znd_hbm, zn_hbm,
             ph_hbm, pd_hbm,
             s_v, d_v, src_r, dst_r, w_c, rows, h_sh, den_sh,
             gsem, hsem, wsem, isem, jsem):
    cid = lax.axis_index("c")
    sid = lax.axis_index("s")
    wid = cid * 16 + sid
    pltpu.sync_copy(s_hbm, s_v)
    pltpu.sync_copy(d_hbm, d_v)
    pltpu.sync_copy(znd_hbm, h_sh.at[pl.ds(sid * ROWS_PT, ROWS_PT)])
    pltpu.sync_copy(zn_hbm, den_sh.at[pl.ds(sid * ROWS_PT, ROWS_PT)])

    ebase = wid * EPW
    pltpu.sync_copy(src_hbm.at[pl.ds(ebase, C)], src_r.at[0])
    pltpu.sync_copy(dst_hbm.at[pl.ds(ebase, C)], dst_r.at[0])
    pltpu.async_copy(src_hbm.at[pl.ds(ebase + C, C)], src_r.at[1], isem.at[1])
    pltpu.async_copy(dst_hbm.at[pl.ds(ebase + C, C)], dst_r.at[1], jsem.at[1])
    pltpu.async_copy(z_hbm.at[src_r.at[0]], rows.at[0], gsem.at[0])

    def slot(t, b, ib, has_next, has_next2):
        bn = 1 - b
        ibn = (ib + 1) % 3
        ibp = (ib + 2) % 3
        pltpu.make_async_copy(z_hbm.at[src_r.at[ib]], rows.at[b],
                              gsem.at[b]).wait()
        if has_next:
            pltpu.make_async_copy(src_hbm.at[pl.ds(ebase + (t + 1) * C, C)], src_r.at[ibn],
                                  isem.at[ibn]).wait()
            pltpu.make_async_copy(dst_hbm.at[pl.ds(ebase + (t + 1) * C, C)], dst_r.at[ibn],
                                  jsem.at[ibn]).wait()

            @pl.when(t >= 1)
            def _():
                pltpu.make_async_copy(rows.at[bn], h_sh.at[dst_r.at[ibp]],
                                      hsem.at[bn]).wait()
                pltpu.make_async_copy(w_c.at[bn], den_sh.at[dst_r.at[ibp]],
                                      wsem.at[bn]).wait()

            pltpu.async_copy(z_hbm.at[src_r.at[ibn]], rows.at[bn],
                             gsem.at[bn])
        for k in range(C // 16):
            si = src_r[ib, pl.ds(k * 16, 16)]
            di = dst_r[ib, pl.ds(k * 16, 16)]
            e = plsc.load_gather(s_v, [si]) + plsc.load_gather(d_v, [di])
            e = jnp.where(e >= 0.0, e, e * 0.01)
            w_c[b, pl.ds(k * 16, 16)] = jnp.exp(e)

        def scale_body(k, c2):
            w16 = w_c[b, pl.ds(k * 16, 16)]
            base16 = k * 16
            for lane in range(16):
                ws = w16[lane]
                i = base16 + lane
                for j in range(D // 16):
                    rows[b, i, pl.ds(j * 16, 16)] = (
                        rows[b, i, pl.ds(j * 16, 16)] * ws)
            return c2

        lax.fori_loop(0, C // 16, scale_body, 0)
        pltpu.async_copy(rows.at[b], h_sh.at[dst_r.at[ib]], hsem.at[b],
                         add=True)
        pltpu.async_copy(w_c.at[b], den_sh.at[dst_r.at[ib]], wsem.at[b],
                         add=True)
        if has_next2:
            pltpu.async_copy(src_hbm.at[pl.ds(ebase + (t + 2) * C, C)], src_r.at[ibp],
                             isem.at[ibp])
            pltpu.async_copy(dst_hbm.at[pl.ds(ebase + (t + 2) * C, C)], dst_r.at[ibp],
                             jsem.at[ibp])

    def group_body(g, carry):
        t0 = g * 6
        for u in range(6):
            slot(t0 + u, u % 2, u % 3, True, True)
        return carry

    plsc.subcore_barrier()
    lax.fori_loop(0, NCHUNK // 6, group_body, 0)
    for t in range(NCHUNK // 6 * 6, NCHUNK):
        slot(t, t % 2, t % 3, t + 1 <= NCHUNK - 1, t + 2 <= NCHUNK - 1)
    for t in (NCHUNK - 2, NCHUNK - 1):
        b, ib = t % 2, t % 3
        pltpu.make_async_copy(rows.at[b], h_sh.at[dst_r.at[ib]],
                              hsem.at[b]).wait()
        pltpu.make_async_copy(w_c.at[b], den_sh.at[dst_r.at[ib]],
                              wsem.at[b]).wait()
    plsc.subcore_barrier()

    pltpu.sync_copy(h_sh.at[pl.ds(sid * ROWS_PT, ROWS_PT)],
                    ph_hbm.at[cid, pl.ds(sid * ROWS_PT, ROWS_PT)])

    @pl.when(sid == 0)
    def _():
        pltpu.sync_copy(den_sh, pd_hbm.at[cid])


def _epilogue_body(ph0_ref, ph1_ref, den_ref, out_ref):
    den = den_ref[0, :] + den_ref[1, :]
    rec = (1.0 / jnp.where(den > 0.0, den, 1.0)).reshape(-1, 1)
    out_ref[...] = (ph0_ref[0] + ph1_ref[0]) * rec


def kernel(x, edge_index, W_fc, W_attn):
    blk = 1024
    grid = NP // blk
    z, s, d = pl.pallas_call(
        _prologue_body,
        grid=(grid,),
        in_specs=[
            pl.BlockSpec((blk, D), lambda i: (i, 0)),
            pl.BlockSpec((D, D), lambda i: (0, 0)),
            pl.BlockSpec((1, 2 * D), lambda i: (0, 0)),
        ],
        out_specs=[
            pl.BlockSpec((blk, D), lambda i: (i, 0)),
            pl.BlockSpec((blk,), lambda i: (i,)),
            pl.BlockSpec((blk,), lambda i: (i,)),
        ],
        out_shape=[
            jax.ShapeDtypeStruct((N, D), jnp.float32),
            jax.ShapeDtypeStruct((N,), jnp.float32),
            jax.ShapeDtypeStruct((N,), jnp.float32),
        ],
    )(x, W_fc, W_attn)

    src = edge_index[0]
    dst = edge_index[1]
    zeros_nd = jnp.zeros((ROWS_PT, D), jnp.float32)
    zeros_n = jnp.zeros((ROWS_PT,), jnp.float32)

    mesh = plsc.VectorSubcoreMesh(core_axis_name="c", subcore_axis_name="s")
    sc = pl.kernel(
        _sc_body,
        out_type=[
            jax.ShapeDtypeStruct((2, NP, D), jnp.float32),
            jax.ShapeDtypeStruct((2, NP), jnp.float32),
        ],
        mesh=mesh,
        scratch_types=[
            pltpu.VMEM((N,), jnp.float32),
            pltpu.VMEM((N,), jnp.float32),
            pltpu.VMEM((3, C), jnp.int32),
            pltpu.VMEM((3, C), jnp.int32),
            pltpu.VMEM((NB, C), jnp.float32),
            pltpu.VMEM((NB, C, D), jnp.float32),
            pltpu.VMEM_SHARED((NP, D), jnp.float32),
            pltpu.VMEM_SHARED((NP,), jnp.float32),
            pltpu.SemaphoreType.DMA((NB,)),
            pltpu.SemaphoreType.DMA((NB,)),
            pltpu.SemaphoreType.DMA((NB,)),
            pltpu.SemaphoreType.DMA((3,)),
            pltpu.SemaphoreType.DMA((3,)),
        ],
        compiler_params=pltpu.CompilerParams(needs_layout_passes=False),
    )
    ph, pd = sc(z, s, d, src, dst, zeros_nd, zeros_n)

    h_pad = pl.pallas_call(
        _epilogue_body,
        grid=(grid,),
        in_specs=[
            pl.BlockSpec((1, blk, D), lambda i: (0, i, 0)),
            pl.BlockSpec((1, blk, D), lambda i: (1, i, 0)),
            pl.BlockSpec((2, blk), lambda i: (0, i)),
        ],
        out_specs=pl.BlockSpec((blk, D), lambda i: (i, 0)),
        out_shape=jax.ShapeDtypeStruct((N, D), jnp.float32),
    )(ph, ph, pd)

    return h_pad


# speedup vs baseline: 1.2907x; 1.0316x over previous
import jax
import jax.numpy as jnp
from jax import lax
from jax.experimental import pallas as pl
from jax.experimental.pallas import tpu as pltpu
from jax.experimental.pallas import tpu_sc as plsc

N = 10000
NP = 10240
D = 128
E = 320000
NW = 32
EPW = E // NW
C = 80
NCHUNK = EPW // C
ROWS_PT = NP // 16
NB = 2


def _prologue_body(x_ref, w_ref, wa_ref, z_ref, s_ref, d_ref):
    z = lax.dot_general(x_ref[...], w_ref[...], (((1,), (1,)), ((), ())),
                        preferred_element_type=jnp.float32)
    z_ref[...] = z
    s_ref[...] = jnp.sum(z * wa_ref[0, :D][None, :], axis=1)
    d_ref[...] = jnp.sum(z * wa_ref[0, D:][None, :], axis=1)


def _sc_body(z_hbm, s_hbm, d_hbm, src_hbm, dst_hbm, znd_hbm, zn_hbm,
             ph_hbm, pd_hbm,
             s_v, d_v, src_r, dst_r, w_c, rows, h_sh, den_sh,
             gsem, hsem, wsem, isem, jsem):
    cid = lax.axis_index("c")
    sid = lax.axis_index("s")
    wid = cid * 16 + sid
    pltpu.sync_copy(s_hbm, s_v)
    pltpu.sync_copy(d_hbm, d_v)
    pltpu.sync_copy(znd_hbm, h_sh.at[pl.ds(sid * ROWS_PT, ROWS_PT)])
    pltpu.sync_copy(zn_hbm, den_sh.at[pl.ds(sid * ROWS_PT, ROWS_PT)])

    ebase = wid * EPW
    pltpu.sync_copy(src_hbm.at[pl.ds(ebase, C)], src_r.at[0])
    pltpu.sync_copy(dst_hbm.at[pl.ds(ebase, C)], dst_r.at[0])
    pltpu.async_copy(src_hbm.at[pl.ds(ebase + C, C)], src_r.at[1], isem.at[1])
    pltpu.async_copy(dst_hbm.at[pl.ds(ebase + C, C)], dst_r.at[1], jsem.at[1])
    pltpu.async_copy(z_hbm.at[src_r.at[0]], rows.at[0], gsem.at[0])

    def slot(t, b, ib, has_next, has_next2):
        bn = 1 - b
        ibn = (ib + 1) % 3
        ibp = (ib + 2) % 3
        pltpu.make_async_copy(z_hbm.at[src_r.at[ib]], rows.at[b],
                              gsem.at[b]).wait()
        if has_next:
            pltpu.make_async_copy(src_hbm.at[pl.ds(ebase + (t + 1) * C, C)],
                                  src_r.at[ibn], isem.at[ibn]).wait()
            pltpu.make_async_copy(dst_hbm.at[pl.ds(ebase + (t + 1) * C, C)],
                                  dst_r.at[ibn], jsem.at[ibn]).wait()

            @pl.when(t >= 1)
            def _():
                pltpu.make_async_copy(rows.at[bn], h_sh.at[dst_r.at[ibp]],
                                      hsem.at[bn]).wait()
                pltpu.make_async_copy(w_c.at[bn], den_sh.at[dst_r.at[ibp]],
                                      wsem.at[bn]).wait()

            pltpu.async_copy(z_hbm.at[src_r.at[ibn]], rows.at[bn],
                             gsem.at[bn])

        def edge_body(k, c2):
            si = src_r[ib, pl.ds(k * 16, 16)]
            di = dst_r[ib, pl.ds(k * 16, 16)]
            e = plsc.load_gather(s_v, [si]) + plsc.load_gather(d_v, [di])
            e = jnp.where(e >= 0.0, e, e * 0.01)
            w = jnp.exp(e)
            w_c[b, pl.ds(k * 16, 16)] = w
            base16 = k * 16
            for lane in range(16):
                ws = w[lane]
                i = base16 + lane
                for j in range(D // 16):
                    rows[b, i, pl.ds(j * 16, 16)] = (
                        rows[b, i, pl.ds(j * 16, 16)] * ws)
            return c2

        lax.fori_loop(0, C // 16, edge_body, 0)
        pltpu.async_copy(rows.at[b], h_sh.at[dst_r.at[ib]], hsem.at[b],
                         add=True)
        pltpu.async_copy(w_c.at[b], den_sh.at[dst_r.at[ib]], wsem.at[b],
                         add=True)
        if has_next2:
            pltpu.async_copy(src_hbm.at[pl.ds(ebase + (t + 2) * C, C)],
                             src_r.at[ibp], isem.at[ibp])
            pltpu.async_copy(dst_hbm.at[pl.ds(ebase + (t + 2) * C, C)],
                             dst_r.at[ibp], jsem.at[ibp])

    def group_body(g, carry):
        t0 = g * 6
        for u in range(6):
            slot(t0 + u, u % 2, u % 3, True, True)
        return carry

    plsc.subcore_barrier()
    lax.fori_loop(0, NCHUNK // 6, group_body, 0)
    for t in range(NCHUNK // 6 * 6, NCHUNK):
        slot(t, t % 2, t % 3, t + 1 <= NCHUNK - 1, t + 2 <= NCHUNK - 1)
    for t in (NCHUNK - 2, NCHUNK - 1):
        b, ib = t % 2, t % 3
        pltpu.make_async_copy(rows.at[b], h_sh.at[dst_r.at[ib]],
                              hsem.at[b]).wait()
        pltpu.make_async_copy(w_c.at[b], den_sh.at[dst_r.at[ib]],
                              wsem.at[b]).wait()
    plsc.subcore_barrier()

    pltpu.sync_copy(h_sh.at[pl.ds(sid * ROWS_PT, ROWS_PT)],
                    ph_hbm.at[cid, pl.ds(sid * ROWS_PT, ROWS_PT)])

    @pl.when(sid == 0)
    def _():
        pltpu.sync_copy(den_sh, pd_hbm.at[cid])


def _epilogue_body(ph0_ref, ph1_ref, den_ref, out_ref):
    den = den_ref[0, :] + den_ref[1, :]
    rec = (1.0 / jnp.where(den > 0.0, den, 1.0)).reshape(-1, 1)
    out_ref[...] = (ph0_ref[0] + ph1_ref[0]) * rec


def kernel(x, edge_index, W_fc, W_attn):
    blk = 1024
    grid = NP // blk
    z, s, d = pl.pallas_call(
        _prologue_body,
        grid=(grid,),
        in_specs=[
            pl.BlockSpec((blk, D), lambda i: (i, 0)),
            pl.BlockSpec((D, D), lambda i: (0, 0)),
            pl.BlockSpec((1, 2 * D), lambda i: (0, 0)),
        ],
        out_specs=[
            pl.BlockSpec((blk, D), lambda i: (i, 0)),
            pl.BlockSpec((blk,), lambda i: (i,)),
            pl.BlockSpec((blk,), lambda i: (i,)),
        ],
        out_shape=[
            jax.ShapeDtypeStruct((N, D), jnp.float32),
            jax.ShapeDtypeStruct((N,), jnp.float32),
            jax.ShapeDtypeStruct((N,), jnp.float32),
        ],
    )(x, W_fc, W_attn)

    src = edge_index[0]
    dst = edge_index[1]
    zeros_nd = jnp.zeros((ROWS_PT, D), jnp.float32)
    zeros_n = jnp.zeros((ROWS_PT,), jnp.float32)

    mesh = plsc.VectorSubcoreMesh(core_axis_name="c", subcore_axis_name="s")
    sc = pl.kernel(
        _sc_body,
        out_type=[
            jax.ShapeDtypeStruct((2, NP, D), jnp.float32),
            jax.ShapeDtypeStruct((2, NP), jnp.float32),
        ],
        mesh=mesh,
        scratch_types=[
            pltpu.VMEM((N,), jnp.float32),
            pltpu.VMEM((N,), jnp.float32),
            pltpu.VMEM((3, C), jnp.int32),
            pltpu.VMEM((3, C), jnp.int32),
            pltpu.VMEM((NB, C), jnp.float32),
            pltpu.VMEM((NB, C, D), jnp.float32),
            pltpu.VMEM_SHARED((NP, D), jnp.float32),
            pltpu.VMEM_SHARED((NP,), jnp.float32),
            pltpu.SemaphoreType.DMA((NB,)),
            pltpu.SemaphoreType.DMA((NB,)),
            pltpu.SemaphoreType.DMA((NB,)),
            pltpu.SemaphoreType.DMA((3,)),
            pltpu.SemaphoreType.DMA((3,)),
        ],
        compiler_params=pltpu.CompilerParams(needs_layout_passes=False),
    )
    ph, pd = sc(z, s, d, src, dst, zeros_nd, zeros_n)

    h_pad = pl.pallas_call(
        _epilogue_body,
        grid=(grid,),
        in_specs=[
            pl.BlockSpec((1, blk, D), lambda i: (0, i, 0)),
            pl.BlockSpec((1, blk, D), lambda i: (1, i, 0)),
            pl.BlockSpec((2, blk), lambda i: (0, i)),
        ],
        out_specs=pl.BlockSpec((blk, D), lambda i: (i, 0)),
        out_shape=jax.ShapeDtypeStruct((N, D), jnp.float32),
    )(ph, ph, pd)

    return h_pad
